# Initial kernel scaffold; baseline (speedup 1.0000x reference)
import functools

import jax
import jax.numpy as jnp
from jax import lax
from jax.experimental import pallas as pl
from jax.experimental.pallas import tpu as pltpu
from jax.experimental.pallas import tpu_sc as plsc

_N = 10000
_D = 128
_E = 320000
_NC = 2
_NS = 16
_EPT = _E // (_NC * _NS)
_CHUNK = 80
_NCHUNK = _EPT // _CHUNK
_RPT = _N // _NS
_ZROWS = 125
_NZ = _RPT // _ZROWS


def _agg_body(x_hbm, src_hbm, dst_hbm, out_hbm,
              acc_sh, src_v, dst_v, rows_v, stage_v, sem):
    c = lax.axis_index("c")
    s = lax.axis_index("s")

    def _zstore(i, _):
        for j in range(_D // 16):
            stage_v[i, pl.ds(j * 16, 16)] = jnp.zeros((16,), jnp.float32)
        return 0

    lax.fori_loop(0, _ZROWS, _zstore, 0)
    for k in range(_NZ):
        row = s * _RPT + k * _ZROWS
        pltpu.sync_copy(stage_v, acc_sh.at[pl.ds(row, _ZROWS)])
    plsc.subcore_barrier()

    ebase = (c * _NS + s) * _EPT

    def _edge_chunk(i, _):
        off = ebase + i * _CHUNK
        pltpu.sync_copy(src_hbm.at[pl.ds(off, _CHUNK)], src_v)
        pltpu.sync_copy(dst_hbm.at[pl.ds(off, _CHUNK)], dst_v)
        pltpu.async_copy(x_hbm.at[src_v], rows_v, sem).wait()
        pltpu.sync_copy(rows_v, acc_sh.at[dst_v], add=True)
        return 0

    lax.fori_loop(0, _NCHUNK, _edge_chunk, 0)
    plsc.subcore_barrier()

    for k in range(_NZ):
        row = s * _RPT + k * _ZROWS
        pltpu.sync_copy(acc_sh.at[pl.ds(row, _ZROWS)], stage_v)
        pltpu.sync_copy(stage_v, out_hbm.at[pl.ds(c * _N + row, _ZROWS)])


_agg = pl.kernel(
    _agg_body,
    out_type=jax.ShapeDtypeStruct((_NC * _N, _D), jnp.float32),
    mesh=plsc.VectorSubcoreMesh(core_axis_name="c", subcore_axis_name="s"),
    scratch_types=[
        pltpu.VMEM_SHARED((_N, _D), jnp.float32),
        pltpu.VMEM((_CHUNK,), jnp.int32),
        pltpu.VMEM((_CHUNK,), jnp.int32),
        pltpu.VMEM((_CHUNK, _D), jnp.float32),
        pltpu.VMEM((_ZROWS, _D), jnp.float32),
        pltpu.SemaphoreType.DMA,
    ],
)


def _mm_body(relu, p0_ref, p1_ref, x_ref, wrelT_ref, wrootT_ref, b_ref, o_ref):
    agg = p0_ref[...] + p1_ref[...]
    out = jnp.dot(agg, wrelT_ref[...],
                  preferred_element_type=jnp.float32,
                  precision=lax.Precision.HIGHEST)
    out = out + jnp.dot(x_ref[...], wrootT_ref[...],
                        preferred_element_type=jnp.float32,
                        precision=lax.Precision.HIGHEST)
    out = out + b_ref[...]
    if relu:
        out = jnp.maximum(out, 0.0)
    o_ref[...] = out


def _mm(p0, p1, x, wrelT, wrootT, b2d, relu):
    blk = 1000
    return pl.pallas_call(
        functools.partial(_mm_body, relu),
        grid=(_N // blk,),
        in_specs=[
            pl.BlockSpec((blk, _D), lambda i: (i, 0)),
            pl.BlockSpec((blk, _D), lambda i: (i, 0)),
            pl.BlockSpec((blk, _D), lambda i: (i, 0)),
            pl.BlockSpec((_D, _D), lambda i: (0, 0)),
            pl.BlockSpec((_D, _D), lambda i: (0, 0)),
            pl.BlockSpec((1, _D), lambda i: (0, 0)),
        ],
        out_specs=pl.BlockSpec((blk, _D), lambda i: (i, 0)),
        out_shape=jax.ShapeDtypeStruct((_N, _D), jnp.float32),
    )(p0, p1, x, wrelT, wrootT, b2d)


def kernel(x, edge_index, W1_rel, W1_root, b1, W2_rel, W2_root, b2):
    src = edge_index[0]
    dst = edge_index[1]
    p = _agg(x, src, dst)
    h = _mm(p[:_N], p[_N:], x, W1_rel.T, W1_root.T, b1.reshape(1, _D), True)
    p = _agg(h, src, dst)
    return _mm(p[:_N], p[_N:], h, W2_rel.T, W2_root.T, b2.reshape(1, _D), False)


# speedup vs baseline: 4.9653x; 4.9653x over previous
import functools

import jax
import jax.numpy as jnp
from jax import lax
from jax.experimental import pallas as pl
from jax.experimental.pallas import tpu as pltpu
from jax.experimental.pallas import tpu_sc as plsc

_N = 10000
_D = 128
_E = 320000
_NC = 2
_NS = 16
_EPT = _E // (_NC * _NS)
_CHUNK = 80
_NCHUNK = _EPT // _CHUNK
_PIECE = 80
_NPIECE = _N // _PIECE


def _agg_body(x_hbm, src_hbm, dst_hbm, out_hbm,
              acc_sh, src_v, dst_v, rows_v, stage_v, sem):
    c = lax.axis_index("c")
    s = lax.axis_index("s")
    npiece_mine = (_NPIECE + _NS - 1 - s) // _NS

    def _zstore(i, _):
        for j in range(_D // 16):
            stage_v[i, pl.ds(j * 16, 16)] = jnp.zeros((16,), jnp.float32)
        return 0

    lax.fori_loop(0, _PIECE, _zstore, 0)

    def _zpiece(i, _):
        row = (s + i * _NS) * _PIECE
        pltpu.sync_copy(stage_v, acc_sh.at[pl.ds(row, _PIECE)])
        return 0

    lax.fori_loop(0, npiece_mine, _zpiece, 0)
    plsc.subcore_barrier()

    ebase = (c * _NS + s) * _EPT

    def _edge_chunk(i, _):
        off = ebase + i * _CHUNK
        pltpu.sync_copy(src_hbm.at[pl.ds(off, _CHUNK)], src_v)
        pltpu.sync_copy(dst_hbm.at[pl.ds(off, _CHUNK)], dst_v)
        pltpu.async_copy(x_hbm.at[src_v], rows_v, sem).wait()
        pltpu.sync_copy(rows_v, acc_sh.at[dst_v], add=True)
        return 0

    lax.fori_loop(0, _NCHUNK, _edge_chunk, 0)
    plsc.subcore_barrier()

    def _wpiece(i, _):
        row = (s + i * _NS) * _PIECE
        pltpu.sync_copy(acc_sh.at[pl.ds(row, _PIECE)], stage_v)
        pltpu.sync_copy(stage_v, out_hbm.at[pl.ds(c * _N + row, _PIECE)])
        return 0

    lax.fori_loop(0, npiece_mine, _wpiece, 0)


_agg = pl.kernel(
    _agg_body,
    out_type=jax.ShapeDtypeStruct((_NC * _N, _D), jnp.float32),
    mesh=plsc.VectorSubcoreMesh(core_axis_name="c", subcore_axis_name="s"),
    scratch_types=[
        pltpu.VMEM_SHARED((_N, _D), jnp.float32),
        pltpu.VMEM((_CHUNK,), jnp.int32),
        pltpu.VMEM((_CHUNK,), jnp.int32),
        pltpu.VMEM((_CHUNK, _D), jnp.float32),
        pltpu.VMEM((_PIECE, _D), jnp.float32),
        pltpu.SemaphoreType.DMA,
    ],
)


def _mm_body(relu, p0_ref, p1_ref, x_ref, wrelT_ref, wrootT_ref, b_ref, o_ref):
    agg = p0_ref[...] + p1_ref[...]
    out = jnp.dot(agg, wrelT_ref[...],
                  preferred_element_type=jnp.float32,
                  precision=lax.Precision.HIGHEST)
    out = out + jnp.dot(x_ref[...], wrootT_ref[...],
                        preferred_element_type=jnp.float32,
                        precision=lax.Precision.HIGHEST)
    out = out + b_ref[...]
    if relu:
        out = jnp.maximum(out, 0.0)
    o_ref[...] = out


def _mm(p0, p1, x, wrelT, wrootT, b2d, relu):
    blk = 1000
    return pl.pallas_call(
        functools.partial(_mm_body, relu),
        grid=(_N // blk,),
        in_specs=[
            pl.BlockSpec((blk, _D), lambda i: (i, 0)),
            pl.BlockSpec((blk, _D), lambda i: (i, 0)),
            pl.BlockSpec((blk, _D), lambda i: (i, 0)),
            pl.BlockSpec((_D, _D), lambda i: (0, 0)),
            pl.BlockSpec((_D, _D), lambda i: (0, 0)),
            pl.BlockSpec((1, _D), lambda i: (0, 0)),
        ],
        out_specs=pl.BlockSpec((blk, _D), lambda i: (i, 0)),
        out_shape=jax.ShapeDtypeStruct((_N, _D), jnp.float32),
    )(p0, p1, x, wrelT, wrootT, b2d)


def kernel(x, edge_index, W1_rel, W1_root, b1, W2_rel, W2_root, b2):
    src = edge_index[0]
    dst = edge_index[1]
    p = _agg(x, src, dst)
    h = _mm(p[:_N], p[_N:], x, W1_rel.T, W1_root.T, b1.reshape(1, _D), True)
    p = _agg(h, src, dst)
    return _mm(p[:_N], p[_N:], h, W2_rel.T, W2_root.T, b2.reshape(1, _D), False)
